# baseline (speedup 1.0000x reference)
import functools

import jax
import jax.numpy as jnp
from jax import lax
from jax.experimental import pallas as pl
from jax.experimental.pallas import tpu as pltpu
from jax.experimental.pallas import tpu_sc as plsc

VOCAB = 1000000
DM = 64
DMP = 128
SEQ = 200
BATCH = 4096

NC, NS = 2, 16
NW = NC * NS
ROWS = BATCH * SEQ
RPW = ROWS // NW
BPW = BATCH // NW
G1 = 104
G2 = SEQ - G1
NBUF = 2
LANES = 16
GRP = DM // LANES


def _make_kernel():
    mesh = plsc.VectorSubcoreMesh(core_axis_name="c", subcore_axis_name="s")

    @functools.partial(
        pl.kernel,
        mesh=mesh,
        out_type=jax.ShapeDtypeStruct((BATCH, SEQ, DM), jnp.float32),
        compiler_params=pltpu.CompilerParams(use_tc_tiling_on_sc=False),
        scratch_types=[
            pltpu.VMEM((RPW,), jnp.int32),
            pltpu.VMEM((SEQ, DM), jnp.float32),
            pltpu.VMEM((NBUF, SEQ, DMP), jnp.float32),
            pltpu.SemaphoreType.DMA((NBUF,)),
            pltpu.SemaphoreType.DMA((NBUF,)),
        ],
    )
    def emb_kernel(idx_hbm, table_hbm, pos_hbm, out_hbm,
                   idx_v, pos_v, rows_v, gsem, osem):
        wid = lax.axis_index("s") * NC + lax.axis_index("c")
        base = wid * RPW
        bbase = wid * BPW
        pltpu.sync_copy(idx_hbm.at[pl.ds(base, RPW)], idx_v)
        pltpu.sync_copy(pos_hbm, pos_v)

        def gather1(j, b):
            return pltpu.make_async_copy(
                table_hbm.at[idx_v.at[pl.ds(j * SEQ, G1)]],
                rows_v.at[b, pl.ds(0, G1)], gsem.at[b])

        def gather2(j, b):
            return pltpu.make_async_copy(
                table_hbm.at[idx_v.at[pl.ds(j * SEQ + G1, G2)]],
                rows_v.at[b, pl.ds(G1, G2)], gsem.at[b])

        def outcopy(j, b):
            return pltpu.make_async_copy(
                rows_v.at[b, :, pl.ds(0, DM)], out_hbm.at[bbase + j],
                osem.at[b])

        def add_pos(b):
            @plsc.parallel_loop(0, SEQ, 1, unroll=4)
            def _(r):
                for g in range(GRP):
                    plsc.addupdate(rows_v.at[b, r, pl.ds(g * LANES, LANES)],
                                   pos_v[r, pl.ds(g * LANES, LANES)])

        for b in range(NBUF):
            gather1(b, b).start()
            gather2(b, b).start()

        def outer(t, carry):
            jj = t * NBUF
            for b in range(NBUF):
                j = jj + b
                gather1(j, b).wait()
                gather2(j, b).wait()
                add_pos(b)
                outcopy(j, b).start()
                jp = j - 1
                bp = (b - 1) % NBUF
                jn = jp + NBUF

                @pl.when((jp >= 0) & (jn < BPW))
                def _():
                    outcopy(jp, bp).wait()
                    gather1(jn, bp).start()
                    gather2(jn, bp).start()

            return carry

        lax.fori_loop(0, BPW // NBUF, outer, 0)
        for k in range(NBUF):
            j = BPW - NBUF + k
            outcopy(j, j % NBUF).wait()

    return emb_kernel


_emb = _make_kernel()


@jax.jit
def kernel(x, emb_table, pos_table):
    flat = x.reshape(-1)
    tab_p = jnp.pad(emb_table, ((0, 0), (0, DMP - DM)))
    return _emb(flat, tab_p, pos_table)


# speedup vs baseline: 1.3202x; 1.3202x over previous
import functools

import jax
import jax.numpy as jnp
from jax import lax
from jax.experimental import pallas as pl
from jax.experimental.pallas import tpu as pltpu
from jax.experimental.pallas import tpu_sc as plsc

VOCAB = 1000000
DM = 64
DMP = 128
SEQ = 200
BATCH = 4096

NC, NS = 2, 16
NW = NC * NS
ROWS = BATCH * SEQ
RPW = ROWS // NW
BPW = BATCH // NW
G1 = 128
G2 = SEQ - G1
NBUF = 2
LANES = 16
GRP = DM // LANES


def _make_kernel():
    mesh = plsc.VectorSubcoreMesh(core_axis_name="c", subcore_axis_name="s")

    @functools.partial(
        pl.kernel,
        mesh=mesh,
        out_type=jax.ShapeDtypeStruct((BATCH, SEQ, DMP), jnp.float32),
        compiler_params=pltpu.CompilerParams(use_tc_tiling_on_sc=False),
        scratch_types=[
            pltpu.VMEM((RPW,), jnp.int32),
            pltpu.VMEM((SEQ, DM), jnp.float32),
            pltpu.VMEM((NBUF, SEQ, DMP), jnp.float32),
            pltpu.SemaphoreType.DMA((NBUF,)),
            pltpu.SemaphoreType.DMA((NBUF,)),
        ],
    )
    def emb_kernel(idx_hbm, table_hbm, pos_hbm, out_hbm,
                   idx_v, pos_v, rows_v, gsem, osem):
        wid = lax.axis_index("s") * NC + lax.axis_index("c")
        base = wid * RPW
        bbase = wid * BPW
        pltpu.sync_copy(idx_hbm.at[pl.ds(base, RPW)], idx_v)
        pltpu.sync_copy(pos_hbm, pos_v)

        def gather1(j, b):
            return pltpu.make_async_copy(
                table_hbm.at[idx_v.at[pl.ds(j * SEQ, G1)]],
                rows_v.at[b, pl.ds(0, G1)], gsem.at[b])

        def gather2(j, b):
            return pltpu.make_async_copy(
                table_hbm.at[idx_v.at[pl.ds(j * SEQ + G1, G2)]],
                rows_v.at[b, pl.ds(G1, G2)], gsem.at[b])

        def outcopy(j, b):
            return pltpu.make_async_copy(
                rows_v.at[b], out_hbm.at[bbase + j], osem.at[b])

        def add_pos(b):
            @plsc.parallel_loop(0, SEQ, 1, unroll=4)
            def _(r):
                for g in range(GRP):
                    plsc.addupdate(rows_v.at[b, r, pl.ds(g * LANES, LANES)],
                                   pos_v[r, pl.ds(g * LANES, LANES)])

        for b in range(NBUF):
            gather1(b, b).start()
            gather2(b, b).start()

        def outer(t, carry):
            jj = t * NBUF
            for b in range(NBUF):
                j = jj + b
                gather1(j, b).wait()
                gather2(j, b).wait()
                add_pos(b)
                outcopy(j, b).start()
                jp = j - 1
                bp = (b - 1) % NBUF
                jn = jp + NBUF

                @pl.when((jp >= 0) & (jn < BPW))
                def _():
                    outcopy(jp, bp).wait()
                    gather1(jn, bp).start()
                    gather2(jn, bp).start()

            return carry

        lax.fori_loop(0, BPW // NBUF, outer, 0)
        for k in range(NBUF):
            j = BPW - NBUF + k
            outcopy(j, j % NBUF).wait()

    return emb_kernel


_emb = _make_kernel()


@jax.jit
def kernel(x, emb_table, pos_table):
    flat = x.reshape(-1)
    tab_p = jnp.pad(emb_table, ((0, 0), (0, DMP - DM)))
    out_p = _emb(flat, tab_p, pos_table)
    return out_p[:, :, :DM]


# speedup vs baseline: 1.7251x; 1.3067x over previous
import functools

import jax
import jax.numpy as jnp
from jax import lax
from jax.experimental import pallas as pl
from jax.experimental.pallas import tpu as pltpu
from jax.experimental.pallas import tpu_sc as plsc

VOCAB = 1000000
DM = 64
DMP = 128
SEQ = 200
BATCH = 4096

NC, NS = 2, 16
NW = NC * NS
ROWS = BATCH * SEQ
RPW = ROWS // NW
BPW = BATCH // NW
G1 = 128
G2 = SEQ - G1
NBUF = 4
LANES = 16
GRP = DM // LANES


def _make_kernel():
    mesh = plsc.VectorSubcoreMesh(core_axis_name="c", subcore_axis_name="s")

    @functools.partial(
        pl.kernel,
        mesh=mesh,
        out_type=jax.ShapeDtypeStruct((BATCH, SEQ, DMP), jnp.float32),
        compiler_params=pltpu.CompilerParams(use_tc_tiling_on_sc=False),
        scratch_types=[
            pltpu.VMEM((RPW,), jnp.int32),
            pltpu.VMEM((SEQ, DM), jnp.float32),
            pltpu.VMEM((NBUF, SEQ, DM), jnp.float32),
            pltpu.SemaphoreType.DMA((NBUF,)),
            pltpu.SemaphoreType.DMA((NBUF,)),
        ],
    )
    def emb_kernel(idx_hbm, table_hbm, pos_hbm, out_hbm,
                   idx_v, pos_v, rows_v, gsem, osem):
        wid = lax.axis_index("s") * NC + lax.axis_index("c")
        base = wid * RPW
        bbase = wid * BPW
        pltpu.sync_copy(idx_hbm.at[pl.ds(base, RPW)], idx_v)
        pltpu.sync_copy(pos_hbm, pos_v)

        def gather1(j, b):
            return pltpu.make_async_copy(
                table_hbm.at[idx_v.at[pl.ds(j * SEQ, G1)]],
                rows_v.at[b, pl.ds(0, G1)], gsem.at[b])

        def gather2(j, b):
            return pltpu.make_async_copy(
                table_hbm.at[idx_v.at[pl.ds(j * SEQ + G1, G2)]],
                rows_v.at[b, pl.ds(G1, G2)], gsem.at[b])

        def outcopy(j, b):
            return pltpu.make_async_copy(
                rows_v.at[b], out_hbm.at[bbase + j, :, pl.ds(0, DM)],
                osem.at[b])

        def add_pos(b):
            @plsc.parallel_loop(0, SEQ, 1, unroll=4)
            def _(r):
                for g in range(GRP):
                    plsc.addupdate(rows_v.at[b, r, pl.ds(g * LANES, LANES)],
                                   pos_v[r, pl.ds(g * LANES, LANES)])

        for b in range(NBUF):
            gather1(b, b).start()
            gather2(b, b).start()

        def outer(t, carry):
            jj = t * NBUF
            for b in range(NBUF):
                j = jj + b
                gather1(j, b).wait()
                gather2(j, b).wait()
                add_pos(b)
                outcopy(j, b).start()
                jp = j - 1
                bp = (b - 1) % NBUF
                jn = jp + NBUF

                @pl.when((jp >= 0) & (jn < BPW))
                def _():
                    outcopy(jp, bp).wait()
                    gather1(jn, bp).start()
                    gather2(jn, bp).start()

            return carry

        lax.fori_loop(0, BPW // NBUF, outer, 0)
        for k in range(NBUF):
            j = BPW - NBUF + k
            outcopy(j, j % NBUF).wait()

    return emb_kernel


_emb = _make_kernel()


@jax.jit
def kernel(x, emb_table, pos_table):
    flat2 = x.reshape(-1).astype(jnp.int32) * 2
    tab_p = jnp.pad(emb_table, ((0, 0), (0, DMP - DM))).reshape(2 * VOCAB, DM)
    out_p = _emb(flat2, tab_p, pos_table)
    return out_p[:, :, :DM]
